# baseline (speedup 1.0000x reference)
import functools

import jax
import jax.numpy as jnp
from jax import lax
from jax.experimental import pallas as pl
from jax.experimental.pallas import tpu as pltpu

B = 1024
T = 4096
FFN = 2048
HID = 8192

HBLK = 256
NH = HID // HBLK


def _mlp_body(x_ref, w1_ref, w3_ref, w2_ref, rw_ref, out_ref):
    j = pl.program_id(0)
    x = x_ref[...]
    w1 = w1_ref[...].astype(jnp.bfloat16)
    w3 = w3_ref[...].astype(jnp.bfloat16)
    h1 = lax.dot_general(x, w1, (((1,), (1,)), ((), ())),
                         preferred_element_type=jnp.float32)
    h3 = lax.dot_general(x, w3, (((1,), (1,)), ((), ())),
                         preferred_element_type=jnp.float32)
    g = (h1 * jax.nn.sigmoid(h1)) * h3
    w2 = w2_ref[...].astype(jnp.bfloat16)
    contrib = lax.dot_general(g.astype(jnp.bfloat16), w2,
                              (((1,), (1,)), ((), ())),
                              preferred_element_type=jnp.float32)

    @pl.when(j == 0)
    def _():
        out_ref[...] = contrib

    @pl.when(j > 0)
    def _():
        out_ref[...] += contrib

    @pl.when(j == NH - 1)
    def _():
        out_ref[...] *= rw_ref[...]


def _mlp(x_bf16, routing_weights, W1, W3, W2, interpret=False):
    return pl.pallas_call(
        _mlp_body,
        grid=(NH,),
        in_specs=[
            pl.BlockSpec((B, FFN), lambda j: (0, 0)),
            pl.BlockSpec((HBLK, FFN), lambda j: (j, 0)),
            pl.BlockSpec((HBLK, FFN), lambda j: (j, 0)),
            pl.BlockSpec((FFN, HBLK), lambda j: (0, j)),
            pl.BlockSpec((B, 1), lambda j: (0, 0)),
        ],
        out_specs=pl.BlockSpec((B, FFN), lambda j: (0, 0)),
        out_shape=jax.ShapeDtypeStruct((B, FFN), jnp.float32),
        compiler_params=pltpu.CompilerParams(
            dimension_semantics=("arbitrary",),
        ),
        interpret=interpret,
    )(x_bf16, W1, W3, W2, routing_weights)


def kernel(hidden_states, input_idx, routing_weights, W1, W2, W3):
    x = jnp.take(hidden_states, input_idx, axis=0)
    return _mlp(x.astype(jnp.bfloat16), routing_weights, W1, W3, W2)
